# baseline (speedup 1.0000x reference)
import dataclasses
import functools

import jax
import jax.numpy as jnp
from jax import lax
from jax.experimental import pallas as pl
from jax.experimental.pallas import tpu as pltpu
from jax.experimental.pallas import tpu_sc as plsc

EMBED_DIM = 32
BATCH = 16384
WEIGHT_DECAY = 1e-4

NW = 32
NIDX = 3 * BATCH
RANGE = 32768
CHUNK = 512
NCH = RANGE // CHUNK
CAP = 64
NBINS = 2 * NCH
SINK = NIDX
OUT_ROWS = NIDX + 128
LAST_WIN = 999552


def _sc_compiler_params():
    cp = pltpu.CompilerParams()
    if "needs_layout_passes" in pltpu.CompilerParams.__dataclass_fields__:
        cp = dataclasses.replace(cp, needs_layout_passes=False)
    return cp


def _sc_gather_all(ut, it, idx_all):
    mesh = plsc.VectorSubcoreMesh(core_axis_name="c", subcore_axis_name="s")

    @functools.partial(
        pl.kernel,
        out_type=jax.ShapeDtypeStruct((OUT_ROWS, 128), jnp.float32),
        mesh=mesh,
        compiler_params=_sc_compiler_params(),
        scratch_types=[
            pltpu.VMEM((NIDX,), jnp.int32),
            pltpu.VMEM((NBINS * CAP,), jnp.int32),
            pltpu.VMEM((NBINS,), jnp.int32),
            pltpu.VMEM((2, EMBED_DIM, CHUNK), jnp.float32),
            pltpu.VMEM((2, 16, 128), jnp.float32),
            pltpu.VMEM((2, 16), jnp.int32),
            pltpu.SemaphoreType.DMA((2,)),
            pltpu.SemaphoreType.DMA((2,)),
        ],
    )
    def gk(ut_hbm, it_hbm, idx_hbm, out_hbm, idx_v, bins_v, cnt_v,
           chunk_v, stage_v, posv_v, csem, osem):
        wid = lax.axis_index("s") * 2 + lax.axis_index("c")
        w_base = wid * RANGE
        iota16 = lax.iota(jnp.int32, 16)
        zeros16 = jnp.zeros((16,), jnp.int32)

        pltpu.sync_copy(idx_hbm, idx_v)

        @pl.loop(0, NBINS // 16)
        def _(k):
            cnt_v[pl.ds(k * 16, 16)] = zeros16

        @pl.loop(0, NBINS * CAP // 16)
        def _(k):
            bins_v[pl.ds(k * 16, 16)] = zeros16 - 1

        @pl.loop(0, NIDX // 16)
        def _(g):
            x = idx_v[pl.ds(g * 16, 16)]
            pos = g * 16 + iota16
            m = (x >> 15) == wid
            local = x & (RANGE - 1)
            tbl = (pos >= BATCH).astype(jnp.int32)
            b = (local >> 9) + tbl * NCH
            pack = (local << 16) | pos
            dup, last = plsc.scan_count(b, mask=m)
            base = plsc.load_gather(cnt_v, [b])
            slot = base + dup - 1
            slot = jnp.minimum(slot, CAP - 1)
            plsc.store_scatter(bins_v, [b * CAP + slot], pack, mask=m)
            plsc.store_scatter(cnt_v, [b], slot + 1, mask=m & last)

        for sb in range(2):
            posv_v[sb] = jnp.full((16,), SINK, jnp.int32)
            pltpu.async_copy(stage_v.at[sb], out_hbm.at[posv_v.at[sb]],
                             osem.at[sb])

        def chunk_base(cc):
            ch = lax.rem(cc, NCH)
            return pl.multiple_of(
                jnp.minimum(w_base + ch * CHUNK, LAST_WIN), 128)

        def issue_fetch(cc, buf):
            @pl.when(cc < NCH)
            def _():
                pltpu.async_copy(
                    ut_hbm.at[:, pl.ds(chunk_base(cc), CHUNK)],
                    chunk_v.at[buf], csem.at[buf])

            @pl.when(jnp.logical_and(cc >= NCH, cc < 2 * NCH))
            def _():
                pltpu.async_copy(
                    it_hbm.at[:, pl.ds(chunk_base(cc), CHUNK)],
                    chunk_v.at[buf], csem.at[buf])

        issue_fetch(jnp.int32(0), 0)

        @pl.loop(0, NCH, step=1)
        def _(half):
            for b in range(2):
                cc = half * 2 + b
                issue_fetch(cc + 1, 1 - b)
                pltpu.make_async_copy(
                    ut_hbm.at[:, pl.ds(0, CHUNK)],
                    chunk_v.at[b], csem.at[b]).wait()
                sh = chunk_base(cc) - w_base
                for g in range(4):
                    sb = g % 2
                    pltpu.make_async_copy(
                        out_hbm.at[pl.ds(0, 16)],
                        stage_v.at[sb], osem.at[sb]).wait()
                    h = bins_v[pl.ds((cc * 4 + g) * 16, 16)]
                    vm = h >= 0
                    hpos = h & 0xFFFF
                    loc = (h >> 16) - sh
                    loc = jnp.where(vm, loc, 0)
                    posv_v[sb] = jnp.where(vm, hpos, SINK)

                    for j in range(EMBED_DIM):
                        j16 = jnp.full((16,), j, jnp.int32)
                        vals = plsc.load_gather(chunk_v.at[b], [j16, loc])
                        plsc.store_scatter(stage_v.at[sb], [iota16, j16],
                                           vals)

                    pltpu.async_copy(stage_v.at[sb],
                                     out_hbm.at[posv_v.at[sb]],
                                     osem.at[sb])

        for sb in range(2):
            pltpu.make_async_copy(out_hbm.at[pl.ds(0, 16)],
                                  stage_v.at[sb], osem.at[sb]).wait()

    return gk(ut, it, idx_all)


def _loss_body(u_ref, p_ref, n_ref, o_ref, acc_ref):
    lane = lax.broadcasted_iota(jnp.int32, (4096, 128), 1)
    m = (lane < EMBED_DIM).astype(jnp.float32)
    u = u_ref[...] * m
    p = p_ref[...] * m
    n = n_ref[...] * m
    sq = jnp.sum(u * u) + jnp.sum(p * p) + jnp.sum(n * n)
    x = jnp.sum(u * p, axis=1) - jnp.sum(u * n, axis=1)
    sp = jnp.sum(jnp.maximum(-x, 0.0) + jnp.log1p(jnp.exp(-jnp.abs(x))))

    @pl.when(pl.program_id(0) == 0)
    def _():
        acc_ref[0] = 0.0
        acc_ref[1] = 0.0

    acc_ref[0] += sq
    acc_ref[1] += sp

    @pl.when(pl.program_id(0) == pl.num_programs(0) - 1)
    def _():
        reg = 0.5 * acc_ref[0] / BATCH
        o_ref[...] = jnp.reshape(acc_ref[1] / BATCH + WEIGHT_DECAY * reg,
                                 (1, 1))


def kernel(users, positive_items, negative_items, user_embedding,
           item_embedding):
    idx_all = jnp.concatenate([users.astype(jnp.int32),
                               positive_items.astype(jnp.int32),
                               negative_items.astype(jnp.int32)])
    g = _sc_gather_all(user_embedding.T, item_embedding.T, idx_all)
    nblk = BATCH // 4096
    out = pl.pallas_call(
        _loss_body,
        grid=(nblk,),
        in_specs=[
            pl.BlockSpec((4096, 128), lambda bk: (bk, 0)),
            pl.BlockSpec((4096, 128), lambda bk: (bk + nblk, 0)),
            pl.BlockSpec((4096, 128), lambda bk: (bk + 2 * nblk, 0)),
        ],
        out_specs=pl.BlockSpec((1, 1), lambda bk: (0, 0)),
        out_shape=jax.ShapeDtypeStruct((1, 1), jnp.float32),
        scratch_shapes=[pltpu.SMEM((2,), jnp.float32)],
    )(g, g, g)
    return out[0, 0]
